# Initial kernel scaffold; baseline (speedup 1.0000x reference)
import jax
import jax.numpy as jnp
from jax.experimental import pallas as pl


def kernel(x, edge_index, W1, att_src1, att_dst1, b1, W2, att_src2, att_dst2, b2):
    raise NotImplementedError("write your pallas kernel here")


# speedup vs baseline: 38.9687x; 38.9687x over previous
import functools

import jax
import jax.numpy as jnp
import numpy as np
from jax import lax
from jax.experimental import pallas as pl
from jax.experimental.pallas import tpu as pltpu
from jax.experimental.pallas import tpu_sc as plsc

N = 10000
D_IN = 1433
H1, C1 = 10, 8
F1 = H1 * C1
SRCW1 = 96
DSTW1 = 16
W2COLS = 16
NACC = 10112
ROWS_PER_TILE = NACC // 16
ZROWS = ROWS_PER_TILE // 4
E_LOOPED = 160000 + N
CHUNK = 128
NWORK = 32
CH_PER_W = 42
EPAD = NWORK * CHUNK * CH_PER_W
EW = EPAD // NWORK
BLK = 128
GRID = (N + BLK - 1) // BLK
F32 = jnp.float32


def _tca_body(x_ref, ws_ref, wd_ref, ts_ref, td_ref):
    xb = x_ref[...]
    ts_ref[...] = jnp.dot(xb, ws_ref[...], preferred_element_type=F32)
    td_ref[...] = jnp.dot(xb, wd_ref[...], preferred_element_type=F32)


def _tc_a(x, ws1, wd1):
    return pl.pallas_call(
        _tca_body,
        grid=(GRID,),
        in_specs=[
            pl.BlockSpec((BLK, D_IN), lambda i: (i, 0)),
            pl.BlockSpec((D_IN, SRCW1), lambda i: (0, 0)),
            pl.BlockSpec((D_IN, DSTW1), lambda i: (0, 0)),
        ],
        out_specs=[
            pl.BlockSpec((BLK, SRCW1), lambda i: (i, 0)),
            pl.BlockSpec((BLK, DSTW1), lambda i: (i, 0)),
        ],
        out_shape=[
            jax.ShapeDtypeStruct((N, SRCW1), F32),
            jax.ShapeDtypeStruct((N, DSTW1), F32),
        ],
    )(x, ws1, wd1)


_MESH = plsc.VectorSubcoreMesh(core_axis_name="c", subcore_axis_name="s")


def _zero_shared(zbuf, acc, sid, width):
    zero = jnp.zeros((16,), F32)

    def zrow(r, carry):
        for k in range(width // 16):
            zbuf[r, pl.ds(16 * k, 16)] = zero
        return carry

    lax.fori_loop(0, ZROWS, zrow, 0)
    for q in range(4):
        pltpu.sync_copy(zbuf, acc.at[pl.ds(sid * ROWS_PER_TILE + q * ZROWS, ZROWS), :])


@functools.partial(
    pl.kernel,
    mesh=_MESH,
    out_type=jax.ShapeDtypeStruct((2, NACC, SRCW1), F32),
    scratch_types=[
        pltpu.VMEM((CHUNK,), jnp.int32),
        pltpu.VMEM((CHUNK,), jnp.int32),
        pltpu.VMEM((CHUNK, SRCW1), F32),
        pltpu.VMEM((CHUNK, DSTW1), F32),
        pltpu.VMEM((CHUNK, SRCW1), F32),
        pltpu.VMEM((ZROWS, SRCW1), F32),
        pltpu.VMEM_SHARED((NACC, SRCW1), F32),
        pltpu.SemaphoreType.DMA,
        pltpu.SemaphoreType.DMA,
    ],
    compiler_params=pltpu.CompilerParams(use_tc_tiling_on_sc=False),
)
def _sc_layer1(src_hbm, dst_hbm, ts_hbm, td_hbm, out_hbm,
               sidx, didx, srcrows, adrows, outrows, zbuf, acc, sem1, sem2):
    cid = lax.axis_index("c")
    sid = lax.axis_index("s")
    wid = cid * 16 + sid
    _zero_shared(zbuf, acc, sid, SRCW1)
    plsc.subcore_barrier()

    ebase = wid * EW

    def chunk_body(g, carry):
        base = ebase + g * CHUNK
        pltpu.sync_copy(src_hbm.at[pl.ds(base, CHUNK)], sidx)
        pltpu.sync_copy(dst_hbm.at[pl.ds(base, CHUNK)], didx)
        cp1 = pltpu.async_copy(ts_hbm.at[sidx], srcrows, sem1)
        cp2 = pltpu.async_copy(td_hbm.at[didx], adrows, sem2)
        cp1.wait()
        cp2.wait()

        def edge_body(e, ecarry):
            asv = srcrows[e, pl.ds(F1, 16)]
            adv = adrows[e, :]
            s = asv + adv
            w = jnp.exp(jnp.where(s > 0, s, 0.2 * s))
            outrows[e, pl.ds(F1, 16)] = w
            lanes = lax.iota(jnp.int32, 16)
            for v in range(5):
                pat = jnp.where(lanes < 8, 2 * v, 2 * v + 1)
                wv = jnp.take_along_axis(w, pat, axis=0)
                hv = srcrows[e, pl.ds(16 * v, 16)]
                outrows[e, pl.ds(16 * v, 16)] = hv * wv
            return ecarry

        lax.fori_loop(0, CHUNK, edge_body, 0)
        pltpu.sync_copy(outrows, acc.at[didx], add=True)
        return carry

    lax.fori_loop(0, CH_PER_W, chunk_body, 0)
    plsc.subcore_barrier()
    pltpu.sync_copy(
        acc.at[pl.ds(sid * ROWS_PER_TILE, ROWS_PER_TILE), :],
        out_hbm.at[cid, pl.ds(sid * ROWS_PER_TILE, ROWS_PER_TILE), :],
    )


@functools.partial(
    pl.kernel,
    mesh=_MESH,
    out_type=jax.ShapeDtypeStruct((2, NACC, W2COLS), F32),
    scratch_types=[
        pltpu.VMEM((CHUNK,), jnp.int32),
        pltpu.VMEM((CHUNK,), jnp.int32),
        pltpu.VMEM((CHUNK, W2COLS), F32),
        pltpu.VMEM((CHUNK, W2COLS), F32),
        pltpu.VMEM((CHUNK, W2COLS), F32),
        pltpu.VMEM((ZROWS, W2COLS), F32),
        pltpu.VMEM_SHARED((NACC, W2COLS), F32),
        pltpu.SemaphoreType.DMA,
        pltpu.SemaphoreType.DMA,
    ],
    compiler_params=pltpu.CompilerParams(use_tc_tiling_on_sc=False),
)
def _sc_layer2(src_hbm, dst_hbm, ts_hbm, td_hbm, out_hbm,
               sidx, didx, srcrows, adrows, outrows, zbuf, acc, sem1, sem2):
    cid = lax.axis_index("c")
    sid = lax.axis_index("s")
    wid = cid * 16 + sid
    _zero_shared(zbuf, acc, sid, W2COLS)
    plsc.subcore_barrier()

    ebase = wid * EW
    lane8 = jnp.full((16,), 8, dtype=jnp.int32)

    def chunk_body(g, carry):
        base = ebase + g * CHUNK
        pltpu.sync_copy(src_hbm.at[pl.ds(base, CHUNK)], sidx)
        pltpu.sync_copy(dst_hbm.at[pl.ds(base, CHUNK)], didx)
        cp1 = pltpu.async_copy(ts_hbm.at[sidx], srcrows, sem1)
        cp2 = pltpu.async_copy(td_hbm.at[didx], adrows, sem2)
        cp1.wait()
        cp2.wait()

        def edge_body(e, ecarry):
            sv = srcrows[e, :]
            s = sv + adrows[e, :]
            w = jnp.exp(jnp.where(s > 0, s, 0.2 * s))
            wb = jnp.take_along_axis(w, lane8, axis=0)
            outrows[e, :] = sv * wb
            return ecarry

        lax.fori_loop(0, CHUNK, edge_body, 0)
        pltpu.sync_copy(outrows, acc.at[didx], add=True)
        return carry

    lax.fori_loop(0, CH_PER_W, chunk_body, 0)
    plsc.subcore_barrier()
    pltpu.sync_copy(
        acc.at[pl.ds(sid * ROWS_PER_TILE, ROWS_PER_TILE), :],
        out_hbm.at[cid, pl.ds(sid * ROWS_PER_TILE, ROWS_PER_TILE), :],
    )


def _tcb_body(p_ref, psel_ref, ex_ref, b1_ref, wc2_ref, s_ref, d_ref, c_ref,
              ts2_ref, td2_ref):
    a = p_ref[0] + p_ref[1]
    msg = a[:, :F1]
    den = jnp.dot(a, psel_ref[...], preferred_element_type=F32)
    rec = 1.0 / (den + 1e-16)
    recx = jnp.dot(rec, ex_ref[...], preferred_element_type=F32)
    h1 = msg * recx + b1_ref[...]
    h1a = jnp.where(h1 > 0, h1, jnp.exp(h1) - 1.0)
    t2 = jnp.dot(h1a, wc2_ref[...], preferred_element_type=F32)
    ts2_ref[...] = jnp.dot(t2, s_ref[...], preferred_element_type=F32) + c_ref[...]
    td2_ref[...] = jnp.dot(t2, d_ref[...], preferred_element_type=F32)


def _tc_b(p1, psel, ex, b1row, wc2, smat, dmat, cvec):
    return pl.pallas_call(
        _tcb_body,
        grid=(GRID,),
        in_specs=[
            pl.BlockSpec((2, BLK, SRCW1), lambda i: (0, i, 0)),
            pl.BlockSpec((SRCW1, H1), lambda i: (0, 0)),
            pl.BlockSpec((H1, F1), lambda i: (0, 0)),
            pl.BlockSpec((1, F1), lambda i: (0, 0)),
            pl.BlockSpec((F1, 9), lambda i: (0, 0)),
            pl.BlockSpec((9, W2COLS), lambda i: (0, 0)),
            pl.BlockSpec((9, W2COLS), lambda i: (0, 0)),
            pl.BlockSpec((1, W2COLS), lambda i: (0, 0)),
        ],
        out_specs=[
            pl.BlockSpec((BLK, W2COLS), lambda i: (i, 0)),
            pl.BlockSpec((BLK, W2COLS), lambda i: (i, 0)),
        ],
        out_shape=[
            jax.ShapeDtypeStruct((N, W2COLS), F32),
            jax.ShapeDtypeStruct((N, W2COLS), F32),
        ],
    )(p1, psel, ex, b1row, wc2, smat, dmat, cvec)


def _tcc_body(p_ref, b2_ref, o_ref):
    a = p_ref[0] + p_ref[1]
    logits = a[:, :7] / (a[:, 7:8] + 1e-16) + b2_ref[...]
    m = jnp.max(logits, axis=-1, keepdims=True)
    z = logits - m
    lse = jnp.log(jnp.sum(jnp.exp(z), axis=-1, keepdims=True))
    o_ref[...] = z - lse


def _tc_c(p2, b2row):
    return pl.pallas_call(
        _tcc_body,
        grid=(GRID,),
        in_specs=[
            pl.BlockSpec((2, BLK, W2COLS), lambda i: (0, i, 0)),
            pl.BlockSpec((1, 7), lambda i: (0, 0)),
        ],
        out_specs=pl.BlockSpec((BLK, 7), lambda i: (i, 0)),
        out_shape=jax.ShapeDtypeStruct((N, 7), F32),
    )(p2, b2row)


def kernel(x, edge_index, W1, att_src1, att_dst1, b1, W2, att_src2, att_dst2, b2):
    eye10 = jnp.eye(H1, dtype=F32)
    As1 = (att_src1[:, :, None] * eye10[:, None, :]).reshape(F1, H1)
    Ad1 = (att_dst1[:, :, None] * eye10[:, None, :]).reshape(F1, H1)
    zpad = jnp.zeros((D_IN, 6), F32)
    ws1 = jnp.concatenate([W1, W1 @ As1, zpad], axis=1)
    wd1 = jnp.concatenate([W1 @ Ad1, zpad], axis=1)

    wc2 = jnp.concatenate(
        [W2, W2 @ att_src2[0][:, None], W2 @ att_dst2[0][:, None]], axis=1
    )

    psel = jnp.asarray(
        np.concatenate([np.zeros((F1, H1)), np.eye(H1), np.zeros((6, H1))], axis=0),
        F32,
    )
    ex = jnp.asarray(np.repeat(np.eye(H1), C1, axis=1), F32)
    smat_np = np.zeros((9, W2COLS), np.float32)
    for j in range(7):
        smat_np[j, j] = 1.0
    smat_np[7, 8:] = 1.0
    smat = jnp.asarray(smat_np)
    dmat_np = np.zeros((9, W2COLS), np.float32)
    dmat_np[8, :] = 1.0
    dmat = jnp.asarray(dmat_np)
    cvec_np = np.zeros((1, W2COLS), np.float32)
    cvec_np[0, 7] = 1.0
    cvec = jnp.asarray(cvec_np)

    loops = jnp.arange(N, dtype=jnp.int32)
    npad = EPAD - E_LOOPED
    src = jnp.concatenate(
        [edge_index[0].astype(jnp.int32), loops, jnp.zeros((npad,), jnp.int32)]
    )
    dst = jnp.concatenate(
        [edge_index[1].astype(jnp.int32), loops, jnp.full((npad,), N, jnp.int32)]
    )

    tsrc1, tdst1 = _tc_a(x, ws1, wd1)
    p1 = _sc_layer1(src, dst, tsrc1, tdst1)
    tsrc2, tdst2 = _tc_b(p1, psel, ex, b1.reshape(1, F1), wc2, smat, dmat, cvec)
    p2 = _sc_layer2(src, dst, tsrc2, tdst2)
    return _tc_c(p2, b2.reshape(1, 7))
